# Initial kernel scaffold; baseline (speedup 1.0000x reference)
import functools

import jax
import jax.numpy as jnp
from jax import lax
from jax.experimental import pallas as pl
from jax.experimental.pallas import tpu as pltpu
from jax.experimental.pallas import tpu_sc as plsc

N = 10000
E = 160000
DIN = 256
DH = 512
DOUT = 512

NC = 2
NS = 16
KW = 128
EPT = E // NS
NWIN = -(-EPT // KW)
EPTP = NWIN * KW
NPAD = EPTP - EPT
PAD_ROWS = 128
NACC = N + PAD_ROWS
RPTZ = NACC // NS
RPT = N // NS


def _make_spmm(W):
    mesh = plsc.VectorSubcoreMesh(core_axis_name="c", subcore_axis_name="s")

    def body(tabA, tabB, gidxA, gidxB, sidxA, sidxB, zeros_hbm, out,
             gi_v, si_v, rows_v, acc_sh, sem):
        cid = lax.axis_index("c")
        sid = lax.axis_index("s")

        def run(tab, gidx, sidx):
            pltpu.sync_copy(zeros_hbm.at[pl.ds(sid * RPTZ, RPTZ)],
                            acc_sh.at[pl.ds(sid * RPTZ, RPTZ)])
            pltpu.sync_copy(gidx.at[sid], gi_v)
            pltpu.sync_copy(sidx.at[sid], si_v)
            plsc.subcore_barrier()

            def win(w, carry):
                pltpu.async_copy(tab.at[gi_v.at[w]], rows_v, sem).wait()
                pltpu.sync_copy(rows_v, acc_sh.at[si_v.at[w]], add=True)
                return carry

            lax.fori_loop(0, NWIN, win, 0)
            plsc.subcore_barrier()
            pltpu.sync_copy(acc_sh.at[pl.ds(sid * RPT, RPT)],
                            out.at[cid, pl.ds(sid * RPT, RPT)])

        @pl.when(cid == 0)
        def _():
            run(tabA, gidxA, sidxA)

        @pl.when(cid == 1)
        def _():
            run(tabB, gidxB, sidxB)

    return pl.kernel(
        body,
        out_type=jax.ShapeDtypeStruct((NC, N, W), jnp.float32),
        mesh=mesh,
        scratch_types=[
            pltpu.VMEM((NWIN, KW), jnp.int32),
            pltpu.VMEM((NWIN, KW), jnp.int32),
            pltpu.VMEM((KW, W), jnp.float32),
            pltpu.VMEM_SHARED((NACC, W), jnp.float32),
            pltpu.SemaphoreType.DMA,
        ],
    )


_spmm16 = _make_spmm(16)
_spmm128 = _make_spmm(128)


def _pad_idx(a, pad_vals):
    a = a.reshape(NS, EPT)
    pad = jnp.broadcast_to(pad_vals[None, :], (NS, NPAD))
    return jnp.concatenate([a, pad], axis=1).reshape(NS, NWIN, KW)


_PREC = lax.Precision.HIGHEST

TBP = 2000
TB = 1000


def _prep_body(x1_ref, x2_ref, deg_ref, y1_ref, y2_ref, dinv16_ref):
    d1 = lax.rsqrt(deg_ref[0][:, 0:1] + 1.0)
    d2 = lax.rsqrt(deg_ref[1][:, 0:1] + 1.0)
    x1 = x1_ref[...]
    x2 = x2_ref[...]
    y1_ref[0] = x1[:, :128] * d1
    y1_ref[1] = x1[:, 128:] * d1
    y2_ref[0] = x2[:, :128] * d2
    y2_ref[1] = x2[:, 128:] * d2
    dinv16_ref[0] = jnp.broadcast_to(d1, (TBP, 16))
    dinv16_ref[1] = jnp.broadcast_to(d2, (TBP, 16))


_prep = pl.pallas_call(
    _prep_body,
    grid=(N // TBP,),
    in_specs=[
        pl.BlockSpec((TBP, DIN), lambda i: (i, 0)),
        pl.BlockSpec((TBP, DIN), lambda i: (i, 0)),
        pl.BlockSpec((2, TBP, 16), lambda i: (0, i, 0)),
    ],
    out_specs=[
        pl.BlockSpec((2, TBP, 128), lambda i: (0, i, 0)),
        pl.BlockSpec((2, TBP, 128), lambda i: (0, i, 0)),
        pl.BlockSpec((2, TBP, 16), lambda i: (0, i, 0)),
    ],
    out_shape=[
        jax.ShapeDtypeStruct((2, N, 128), jnp.float32),
        jax.ShapeDtypeStruct((2, N, 128), jnp.float32),
        jax.ShapeDtypeStruct((2, N, 16), jnp.float32),
    ],
)


def _mm1_body(p1acc_ref, y_ref, dinv16_ref, W1_ref, b1_ref, out_ref):
    dinv = dinv16_ref[:, 0:1]
    lo = (p1acc_ref[0] + y_ref[0]) * dinv
    hi = (p1acc_ref[1] + y_ref[1]) * dinv
    P1 = jnp.concatenate([lo, hi], axis=1)
    H1 = jnp.dot(P1, W1_ref[...], preferred_element_type=jnp.float32,
                 precision=_PREC) + b1_ref[...]
    y2 = jnp.maximum(H1, 0.0) * dinv
    out_ref[0] = y2[:, 0:128]
    out_ref[1] = y2[:, 128:256]
    out_ref[2] = y2[:, 256:384]
    out_ref[3] = y2[:, 384:512]


_mm1 = pl.pallas_call(
    _mm1_body,
    grid=(N // TB,),
    in_specs=[
        pl.BlockSpec((2, TB, 128), lambda i: (0, i, 0)),
        pl.BlockSpec((2, TB, 128), lambda i: (0, i, 0)),
        pl.BlockSpec((TB, 16), lambda i: (i, 0)),
        pl.BlockSpec((DIN, DH), lambda i: (0, 0)),
        pl.BlockSpec((1, DH), lambda i: (0, 0)),
    ],
    out_specs=pl.BlockSpec((4, TB, 128), lambda i: (0, i, 0)),
    out_shape=jax.ShapeDtypeStruct((4, N, 128), jnp.float32),
)


def _mm2_body(p2lo_ref, p2hi_ref, y_ref, dinv16_ref, cacc16_ref, W2_ref,
              b2_ref, out_ref):
    i = pl.program_id(0)
    dinv = dinv16_ref[:, 0:1]
    P2 = jnp.concatenate([
        (p2lo_ref[0] + y_ref[0]) * dinv,
        (p2lo_ref[1] + y_ref[1]) * dinv,
        (p2hi_ref[0] + y_ref[2]) * dinv,
        (p2hi_ref[1] + y_ref[3]) * dinv,
    ], axis=1)
    H2 = jnp.maximum(
        jnp.dot(P2, W2_ref[...], preferred_element_type=jnp.float32,
                precision=_PREC) + b2_ref[...], 0.0)
    c = dinv * (cacc16_ref[:, 0:1] + dinv)
    part = jnp.sum(H2 * c, axis=0, keepdims=True)

    @pl.when(i == 0)
    def _():
        out_ref[...] = part

    @pl.when(i != 0)
    def _():
        out_ref[...] += part


_mm2 = pl.pallas_call(
    _mm2_body,
    grid=(N // TB,),
    in_specs=[
        pl.BlockSpec((2, TB, 128), lambda i: (0, i, 0)),
        pl.BlockSpec((2, TB, 128), lambda i: (0, i, 0)),
        pl.BlockSpec((4, TB, 128), lambda i: (0, i, 0)),
        pl.BlockSpec((TB, 16), lambda i: (i, 0)),
        pl.BlockSpec((TB, 16), lambda i: (i, 0)),
        pl.BlockSpec((DH, DH), lambda i: (0, 0)),
        pl.BlockSpec((1, DH), lambda i: (0, 0)),
    ],
    out_specs=pl.BlockSpec((1, DH), lambda i: (0, 0)),
    out_shape=jax.ShapeDtypeStruct((1, DH), jnp.float32),
)


def _head_body(s1_ref, s2_ref, W3_ref, b3_ref, Wf1_ref, bf1_ref, Wf2_ref,
               bf2_ref, out_ref):
    r1 = jnp.dot(s1_ref[...] * (1.0 / N), W3_ref[...],
                 preferred_element_type=jnp.float32, precision=_PREC) + b3_ref[...]
    r2 = jnp.dot(s2_ref[...] * (1.0 / N), W3_ref[...],
                 preferred_element_type=jnp.float32, precision=_PREC) + b3_ref[...]
    f = (jnp.dot(r1, Wf1_ref[:DOUT], preferred_element_type=jnp.float32,
                 precision=_PREC)
         + jnp.dot(r2, Wf1_ref[DOUT:], preferred_element_type=jnp.float32,
                   precision=_PREC)
         + bf1_ref[...])
    f = jnp.maximum(f, 0.0)
    out_ref[...] = jnp.tanh(
        jnp.dot(f, Wf2_ref[...], preferred_element_type=jnp.float32,
                precision=_PREC) + bf2_ref[...])


_head = pl.pallas_call(
    _head_body,
    out_shape=jax.ShapeDtypeStruct((1, DOUT), jnp.float32),
)


def kernel(x1, x2, edge_index1, edge_index2, W1, b1, W2, b2, W3, b3,
           Wf1, bf1, Wf2, bf2):
    src1, dst1 = edge_index1[0], edge_index1[1]
    src2, dst2 = edge_index2[0], edge_index2[1]

    ar = jnp.arange(NPAD, dtype=jnp.int32)
    gpad = (ar * 79) % N
    spad = N + (ar % PAD_ROWS)
    src1p = _pad_idx(src1, gpad)
    dst1p = _pad_idx(dst1, spad)
    src2p = _pad_idx(src2, gpad)
    dst2p = _pad_idx(dst2, spad)

    zeros16 = jnp.zeros((NACC, 16), jnp.float32)
    zeros128 = jnp.zeros((NACC, 128), jnp.float32)
    ones16 = jnp.ones((N, 16), jnp.float32)

    deg16 = _spmm16(ones16, ones16, src1p, src2p, dst1p, dst2p, zeros16)

    y1, y2, dinv16 = _prep(x1, x2, deg16)
    dinv16_1, dinv16_2 = dinv16[0], dinv16[1]

    cacc16 = _spmm16(dinv16_1, dinv16_2, dst1p, dst2p, src1p, src2p, zeros16)

    p1a = _spmm128(y1[0], y1[1], src1p, src1p, dst1p, dst1p, zeros128)
    p1b = _spmm128(y2[0], y2[1], src2p, src2p, dst2p, dst2p, zeros128)

    b1r = b1.reshape(1, DH)
    y2nd1 = _mm1(p1a, y1, dinv16_1, W1, b1r)
    y2nd2 = _mm1(p1b, y2, dinv16_2, W1, b1r)

    p2a_lo = _spmm128(y2nd1[0], y2nd1[1], src1p, src1p, dst1p, dst1p, zeros128)
    p2a_hi = _spmm128(y2nd1[2], y2nd1[3], src1p, src1p, dst1p, dst1p, zeros128)
    p2b_lo = _spmm128(y2nd2[0], y2nd2[1], src2p, src2p, dst2p, dst2p, zeros128)
    p2b_hi = _spmm128(y2nd2[2], y2nd2[3], src2p, src2p, dst2p, dst2p, zeros128)

    b2r = b2.reshape(1, DH)
    pooled1 = _mm2(p2a_lo, p2a_hi, y2nd1, dinv16_1, cacc16[0], W2, b2r)
    pooled2 = _mm2(p2b_lo, p2b_hi, y2nd2, dinv16_2, cacc16[1], W2, b2r)

    return _head(pooled1, pooled2, W3, b3.reshape(1, DH), Wf1,
                 bf1.reshape(1, DH), Wf2, bf2.reshape(1, DOUT))


# speedup vs baseline: 11.7574x; 11.7574x over previous
import functools

import jax
import jax.numpy as jnp
from jax import lax
from jax.experimental import pallas as pl
from jax.experimental.pallas import tpu as pltpu
from jax.experimental.pallas import tpu_sc as plsc

N = 10000
E = 160000
DIN = 256
DH = 512
DOUT = 512

NC = 2
NS = 16
KW = 128
EPT = E // NS
NWIN = -(-EPT // KW)
EPTP = NWIN * KW
NPAD = EPTP - EPT
PAD_ROWS = 240
NACC = N + PAD_ROWS
RPTZ = NACC // NS


def _make_spmm(W):
    mesh = plsc.VectorSubcoreMesh(core_axis_name="c", subcore_axis_name="s")

    def body(tabA, tabB, gidxA, gidxB, sidxA, sidxB, zeros_hbm, out,
             gi_v, si_v, rows_v, acc_sh, sem):
        cid = lax.axis_index("c")
        sid = lax.axis_index("s")

        def run(tab, gidx, sidx):
            pltpu.sync_copy(zeros_hbm.at[pl.ds(sid * RPTZ, RPTZ)],
                            acc_sh.at[pl.ds(sid * RPTZ, RPTZ)])
            pltpu.sync_copy(gidx.at[sid], gi_v)
            pltpu.sync_copy(sidx.at[sid], si_v)
            plsc.subcore_barrier()

            def win(w, carry):
                pltpu.async_copy(tab.at[gi_v.at[w]], rows_v, sem).wait()
                pltpu.sync_copy(rows_v, acc_sh.at[si_v.at[w]], add=True)
                return carry

            lax.fori_loop(0, NWIN, win, 0)
            plsc.subcore_barrier()
            pltpu.sync_copy(acc_sh.at[pl.ds(sid * RPTZ, RPTZ)],
                            out.at[cid, pl.ds(sid * RPTZ, RPTZ)])

        @pl.when(cid == 0)
        def _():
            run(tabA, gidxA, sidxA)

        @pl.when(cid == 1)
        def _():
            run(tabB, gidxB, sidxB)

    return pl.kernel(
        body,
        out_type=jax.ShapeDtypeStruct((NC, NACC, W), jnp.float32),
        mesh=mesh,
        scratch_types=[
            pltpu.VMEM((NWIN, KW), jnp.int32),
            pltpu.VMEM((NWIN, KW), jnp.int32),
            pltpu.VMEM((KW, W), jnp.float32),
            pltpu.VMEM_SHARED((NACC, W), jnp.float32),
            pltpu.SemaphoreType.DMA,
        ],
    )


_spmm128 = _make_spmm(128)


def _pad_idx(a, pad_vals):
    a = a.reshape(NS, EPT)
    pad = jnp.broadcast_to(pad_vals[None, :], (NS, NPAD))
    return jnp.concatenate([a, pad], axis=1).reshape(NS, NWIN, KW)


_PREC = lax.Precision.HIGHEST

TBP = 2000
TB = 1000


def _prep_body(x1_ref, x2_ref, deg_ref, y1_ref, y2_ref, dinv_ref):
    d1 = lax.rsqrt(deg_ref[0][:, 0:1] + 1.0)
    d2 = lax.rsqrt(deg_ref[1][:, 0:1] + 1.0)
    x1 = x1_ref[...]
    x2 = x2_ref[...]
    y1_ref[0] = x1[:, :128] * d1
    y1_ref[1] = x1[:, 128:] * d1
    y2_ref[0] = x2[:, :128] * d2
    y2_ref[1] = x2[:, 128:] * d2
    dinv_ref[0] = jnp.broadcast_to(d1, (TBP, 128))
    dinv_ref[1] = jnp.broadcast_to(d2, (TBP, 128))


_prep = pl.pallas_call(
    _prep_body,
    grid=(N // TBP,),
    in_specs=[
        pl.BlockSpec((TBP, DIN), lambda i: (i, 0)),
        pl.BlockSpec((TBP, DIN), lambda i: (i, 0)),
        pl.BlockSpec((2, TBP, 128), lambda i: (0, i, 0)),
    ],
    out_specs=[
        pl.BlockSpec((2, TBP, 128), lambda i: (0, i, 0)),
        pl.BlockSpec((2, TBP, 128), lambda i: (0, i, 0)),
        pl.BlockSpec((2, TBP, 128), lambda i: (0, i, 0)),
    ],
    out_shape=[
        jax.ShapeDtypeStruct((2, N, 128), jnp.float32),
        jax.ShapeDtypeStruct((2, N, 128), jnp.float32),
        jax.ShapeDtypeStruct((2, N, 128), jnp.float32),
    ],
)


def _mm1_body(p1acc_ref, y_ref, dinv_ref, W1_ref, b1_ref, out_ref):
    dinv = dinv_ref[:, 0:1]
    lo = (p1acc_ref[0] + y_ref[0]) * dinv
    hi = (p1acc_ref[1] + y_ref[1]) * dinv
    P1 = jnp.concatenate([lo, hi], axis=1)
    H1 = jnp.dot(P1, W1_ref[...], preferred_element_type=jnp.float32,
                 precision=_PREC) + b1_ref[...]
    y2 = jnp.maximum(H1, 0.0) * dinv
    out_ref[0] = y2[:, 0:128]
    out_ref[1] = y2[:, 128:256]
    out_ref[2] = y2[:, 256:384]
    out_ref[3] = y2[:, 384:512]


_mm1 = pl.pallas_call(
    _mm1_body,
    grid=(N // TB,),
    in_specs=[
        pl.BlockSpec((2, TB, 128), lambda i: (0, i, 0)),
        pl.BlockSpec((2, TB, 128), lambda i: (0, i, 0)),
        pl.BlockSpec((TB, 128), lambda i: (i, 0)),
        pl.BlockSpec((DIN, DH), lambda i: (0, 0)),
        pl.BlockSpec((1, DH), lambda i: (0, 0)),
    ],
    out_specs=pl.BlockSpec((4, TB, 128), lambda i: (0, i, 0)),
    out_shape=jax.ShapeDtypeStruct((4, N, 128), jnp.float32),
)


def _mm2_body(p2lo_ref, p2hi_ref, y_ref, dinv_ref, cacc_ref, W2_ref,
              b2_ref, out_ref):
    i = pl.program_id(0)
    dinv = dinv_ref[:, 0:1]
    P2 = jnp.concatenate([
        (p2lo_ref[0] + y_ref[0]) * dinv,
        (p2lo_ref[1] + y_ref[1]) * dinv,
        (p2hi_ref[0] + y_ref[2]) * dinv,
        (p2hi_ref[1] + y_ref[3]) * dinv,
    ], axis=1)
    H2 = jnp.maximum(
        jnp.dot(P2, W2_ref[...], preferred_element_type=jnp.float32,
                precision=_PREC) + b2_ref[...], 0.0)
    c = dinv * (cacc_ref[:, 0:1] + dinv)
    part = jnp.sum(H2 * c, axis=0, keepdims=True)

    @pl.when(i == 0)
    def _():
        out_ref[...] = part

    @pl.when(i != 0)
    def _():
        out_ref[...] += part


_mm2 = pl.pallas_call(
    _mm2_body,
    grid=(N // TB,),
    in_specs=[
        pl.BlockSpec((2, TB, 128), lambda i: (0, i, 0)),
        pl.BlockSpec((2, TB, 128), lambda i: (0, i, 0)),
        pl.BlockSpec((4, TB, 128), lambda i: (0, i, 0)),
        pl.BlockSpec((TB, 128), lambda i: (i, 0)),
        pl.BlockSpec((TB, 128), lambda i: (i, 0)),
        pl.BlockSpec((DH, DH), lambda i: (0, 0)),
        pl.BlockSpec((1, DH), lambda i: (0, 0)),
    ],
    out_specs=pl.BlockSpec((1, DH), lambda i: (0, 0)),
    out_shape=jax.ShapeDtypeStruct((1, DH), jnp.float32),
)


def _head_body(s1_ref, s2_ref, W3_ref, b3_ref, Wf1_ref, bf1_ref, Wf2_ref,
               bf2_ref, out_ref):
    r1 = jnp.dot(s1_ref[...] * (1.0 / N), W3_ref[...],
                 preferred_element_type=jnp.float32, precision=_PREC) + b3_ref[...]
    r2 = jnp.dot(s2_ref[...] * (1.0 / N), W3_ref[...],
                 preferred_element_type=jnp.float32, precision=_PREC) + b3_ref[...]
    f = (jnp.dot(r1, Wf1_ref[:DOUT], preferred_element_type=jnp.float32,
                 precision=_PREC)
         + jnp.dot(r2, Wf1_ref[DOUT:], preferred_element_type=jnp.float32,
                   precision=_PREC)
         + bf1_ref[...])
    f = jnp.maximum(f, 0.0)
    out_ref[...] = jnp.tanh(
        jnp.dot(f, Wf2_ref[...], preferred_element_type=jnp.float32,
                precision=_PREC) + bf2_ref[...])


_head = pl.pallas_call(
    _head_body,
    out_shape=jax.ShapeDtypeStruct((1, DOUT), jnp.float32),
)


def kernel(x1, x2, edge_index1, edge_index2, W1, b1, W2, b2, W3, b3,
           Wf1, bf1, Wf2, bf2):
    src1, dst1 = edge_index1[0], edge_index1[1]
    src2, dst2 = edge_index2[0], edge_index2[1]

    ar = jnp.arange(NPAD, dtype=jnp.int32)
    gpad = (ar * 79) % N
    spad = N + (ar % PAD_ROWS)
    src1p = _pad_idx(src1, gpad)
    dst1p = _pad_idx(dst1, spad)
    src2p = _pad_idx(src2, gpad)
    dst2p = _pad_idx(dst2, spad)
    dst1g = _pad_idx(dst1, gpad)
    src1s = _pad_idx(src1, spad)
    dst2g = _pad_idx(dst2, gpad)
    src2s = _pad_idx(src2, spad)

    zeros128 = jnp.zeros((NACC, 128), jnp.float32)
    ones128 = jnp.ones((N, 128), jnp.float32)

    deg = _spmm128(ones128, ones128, src1p, src2p, dst1p, dst2p, zeros128)

    y1, y2, dinv = _prep(x1, x2, deg)
    dinv_1, dinv_2 = dinv[0], dinv[1]

    cacc = _spmm128(dinv_1, dinv_2, dst1g, dst2g, src1s, src2s, zeros128)

    p1a = _spmm128(y1[0], y1[1], src1p, src1p, dst1p, dst1p, zeros128)
    p1b = _spmm128(y2[0], y2[1], src2p, src2p, dst2p, dst2p, zeros128)

    b1r = b1.reshape(1, DH)
    y2nd1 = _mm1(p1a, y1, dinv_1, W1, b1r)
    y2nd2 = _mm1(p1b, y2, dinv_2, W1, b1r)

    p2a_lo = _spmm128(y2nd1[0], y2nd1[1], src1p, src1p, dst1p, dst1p, zeros128)
    p2a_hi = _spmm128(y2nd1[2], y2nd1[3], src1p, src1p, dst1p, dst1p, zeros128)
    p2b_lo = _spmm128(y2nd2[0], y2nd2[1], src2p, src2p, dst2p, dst2p, zeros128)
    p2b_hi = _spmm128(y2nd2[2], y2nd2[3], src2p, src2p, dst2p, dst2p, zeros128)

    b2r = b2.reshape(1, DH)
    pooled1 = _mm2(p2a_lo, p2a_hi, y2nd1, dinv_1, cacc[0], W2, b2r)
    pooled2 = _mm2(p2b_lo, p2b_hi, y2nd2, dinv_2, cacc[1], W2, b2r)

    return _head(pooled1, pooled2, W3, b3.reshape(1, DH), Wf1,
                 bf1.reshape(1, DH), Wf2, bf2.reshape(1, DOUT))


# speedup vs baseline: 17.3829x; 1.4785x over previous
import functools

import jax
import jax.numpy as jnp
from jax import lax
from jax.experimental import pallas as pl
from jax.experimental.pallas import tpu as pltpu
from jax.experimental.pallas import tpu_sc as plsc

N = 10000
E = 160000
DIN = 256
DH = 512
DOUT = 512

NC = 2
NS = 16
KW = 128
EPT = E // NS
NWIN = 80
NPH = 2
WPH = NWIN // NPH
EPTP = NWIN * KW
NPAD = EPTP - EPT
PAD_ROWS = 240
NACC = N + PAD_ROWS
RPTZ = NACC // NS


def _make_spmm(W):
    mesh = plsc.VectorSubcoreMesh(core_axis_name="c", subcore_axis_name="s")

    def body(tabA, tabB, gidxA, gidxB, sidxA, sidxB, zeros_hbm, out,
             gi_v, si_v, rows_a, rows_b, acc_sh, sem_a, sem_b):
        cid = lax.axis_index("c")
        sid = lax.axis_index("s")

        def run(tab, gidx, sidx):
            pltpu.sync_copy(zeros_hbm.at[pl.ds(sid * RPTZ, RPTZ)],
                            acc_sh.at[pl.ds(sid * RPTZ, RPTZ)])
            plsc.subcore_barrier()

            for ph in range(NPH):
                pltpu.sync_copy(gidx.at[sid, pl.ds(ph * WPH, WPH)], gi_v)
                pltpu.sync_copy(sidx.at[sid, pl.ds(ph * WPH, WPH)], si_v)

                pltpu.async_copy(tab.at[gi_v.at[0]], rows_a, sem_a)

                def win(p, carry):
                    w0 = 2 * p
                    w1 = w0 + 1
                    pltpu.async_copy(tab.at[gi_v.at[w1]], rows_b, sem_b)
                    pltpu.make_async_copy(tab.at[gi_v.at[w0]], rows_a,
                                          sem_a).wait()
                    pltpu.sync_copy(rows_a, acc_sh.at[si_v.at[w0]], add=True)

                    @pl.when(p < WPH // 2 - 1)
                    def _():
                        pltpu.async_copy(tab.at[gi_v.at[w0 + 2]], rows_a,
                                         sem_a)

                    pltpu.make_async_copy(tab.at[gi_v.at[w1]], rows_b,
                                          sem_b).wait()
                    pltpu.sync_copy(rows_b, acc_sh.at[si_v.at[w1]], add=True)
                    return carry

                lax.fori_loop(0, WPH // 2, win, 0)

            plsc.subcore_barrier()
            pltpu.sync_copy(acc_sh.at[pl.ds(sid * RPTZ, RPTZ)],
                            out.at[cid, pl.ds(sid * RPTZ, RPTZ)])

        @pl.when(cid == 0)
        def _():
            run(tabA, gidxA, sidxA)

        @pl.when(cid == 1)
        def _():
            run(tabB, gidxB, sidxB)

    return pl.kernel(
        body,
        out_type=jax.ShapeDtypeStruct((NC, NACC, W), jnp.float32),
        mesh=mesh,
        scratch_types=[
            pltpu.VMEM((WPH, KW), jnp.int32),
            pltpu.VMEM((WPH, KW), jnp.int32),
            pltpu.VMEM((KW, W), jnp.float32),
            pltpu.VMEM((KW, W), jnp.float32),
            pltpu.VMEM_SHARED((NACC, W), jnp.float32),
            pltpu.SemaphoreType.DMA,
            pltpu.SemaphoreType.DMA,
        ],
    )


_spmm128 = _make_spmm(128)


def _pad_idx(a, pad_vals):
    a = a.reshape(NS, EPT)
    pad = jnp.broadcast_to(pad_vals[None, :], (NS, NPAD))
    return jnp.concatenate([a, pad], axis=1).reshape(NS, NWIN, KW)


_PREC = lax.Precision.HIGHEST

TBP = 2000
TB = 1000


def _prep_body(x1_ref, x2_ref, deg_ref, y1_ref, y2_ref, dinv_ref):
    d1 = lax.rsqrt(deg_ref[0][:, 0:1] + 1.0)
    d2 = lax.rsqrt(deg_ref[1][:, 0:1] + 1.0)
    x1 = x1_ref[...]
    x2 = x2_ref[...]
    y1_ref[0] = x1[:, :128] * d1
    y1_ref[1] = x1[:, 128:] * d1
    y2_ref[0] = x2[:, :128] * d2
    y2_ref[1] = x2[:, 128:] * d2
    dinv_ref[0] = jnp.broadcast_to(d1, (TBP, 128))
    dinv_ref[1] = jnp.broadcast_to(d2, (TBP, 128))


_prep = pl.pallas_call(
    _prep_body,
    grid=(N // TBP,),
    in_specs=[
        pl.BlockSpec((TBP, DIN), lambda i: (i, 0)),
        pl.BlockSpec((TBP, DIN), lambda i: (i, 0)),
        pl.BlockSpec((2, TBP, 128), lambda i: (0, i, 0)),
    ],
    out_specs=[
        pl.BlockSpec((2, TBP, 128), lambda i: (0, i, 0)),
        pl.BlockSpec((2, TBP, 128), lambda i: (0, i, 0)),
        pl.BlockSpec((2, TBP, 128), lambda i: (0, i, 0)),
    ],
    out_shape=[
        jax.ShapeDtypeStruct((2, N, 128), jnp.float32),
        jax.ShapeDtypeStruct((2, N, 128), jnp.float32),
        jax.ShapeDtypeStruct((2, N, 128), jnp.float32),
    ],
)


def _mm1_body(p1acc_ref, y_ref, dinv_ref, W1_ref, b1_ref, out_ref):
    dinv = dinv_ref[:, 0:1]
    lo = (p1acc_ref[0] + y_ref[0]) * dinv
    hi = (p1acc_ref[1] + y_ref[1]) * dinv
    P1 = jnp.concatenate([lo, hi], axis=1)
    H1 = jnp.dot(P1, W1_ref[...], preferred_element_type=jnp.float32,
                 precision=_PREC) + b1_ref[...]
    y2 = jnp.maximum(H1, 0.0) * dinv
    out_ref[0] = y2[:, 0:128]
    out_ref[1] = y2[:, 128:256]
    out_ref[2] = y2[:, 256:384]
    out_ref[3] = y2[:, 384:512]


_mm1 = pl.pallas_call(
    _mm1_body,
    grid=(N // TB,),
    in_specs=[
        pl.BlockSpec((2, TB, 128), lambda i: (0, i, 0)),
        pl.BlockSpec((2, TB, 128), lambda i: (0, i, 0)),
        pl.BlockSpec((TB, 128), lambda i: (i, 0)),
        pl.BlockSpec((DIN, DH), lambda i: (0, 0)),
        pl.BlockSpec((1, DH), lambda i: (0, 0)),
    ],
    out_specs=pl.BlockSpec((4, TB, 128), lambda i: (0, i, 0)),
    out_shape=jax.ShapeDtypeStruct((4, N, 128), jnp.float32),
)


def _mm2_body(p2lo_ref, p2hi_ref, y_ref, dinv_ref, cacc_ref, W2_ref,
              b2_ref, out_ref):
    i = pl.program_id(0)
    dinv = dinv_ref[:, 0:1]
    P2 = jnp.concatenate([
        (p2lo_ref[0] + y_ref[0]) * dinv,
        (p2lo_ref[1] + y_ref[1]) * dinv,
        (p2hi_ref[0] + y_ref[2]) * dinv,
        (p2hi_ref[1] + y_ref[3]) * dinv,
    ], axis=1)
    H2 = jnp.maximum(
        jnp.dot(P2, W2_ref[...], preferred_element_type=jnp.float32,
                precision=_PREC) + b2_ref[...], 0.0)
    c = dinv * (cacc_ref[:, 0:1] + dinv)
    part = jnp.sum(H2 * c, axis=0, keepdims=True)

    @pl.when(i == 0)
    def _():
        out_ref[...] = part

    @pl.when(i != 0)
    def _():
        out_ref[...] += part


_mm2 = pl.pallas_call(
    _mm2_body,
    grid=(N // TB,),
    in_specs=[
        pl.BlockSpec((2, TB, 128), lambda i: (0, i, 0)),
        pl.BlockSpec((2, TB, 128), lambda i: (0, i, 0)),
        pl.BlockSpec((4, TB, 128), lambda i: (0, i, 0)),
        pl.BlockSpec((TB, 128), lambda i: (i, 0)),
        pl.BlockSpec((TB, 128), lambda i: (i, 0)),
        pl.BlockSpec((DH, DH), lambda i: (0, 0)),
        pl.BlockSpec((1, DH), lambda i: (0, 0)),
    ],
    out_specs=pl.BlockSpec((1, DH), lambda i: (0, 0)),
    out_shape=jax.ShapeDtypeStruct((1, DH), jnp.float32),
)


def _head_body(s1_ref, s2_ref, W3_ref, b3_ref, Wf1_ref, bf1_ref, Wf2_ref,
               bf2_ref, out_ref):
    r1 = jnp.dot(s1_ref[...] * (1.0 / N), W3_ref[...],
                 preferred_element_type=jnp.float32, precision=_PREC) + b3_ref[...]
    r2 = jnp.dot(s2_ref[...] * (1.0 / N), W3_ref[...],
                 preferred_element_type=jnp.float32, precision=_PREC) + b3_ref[...]
    f = (jnp.dot(r1, Wf1_ref[:DOUT], preferred_element_type=jnp.float32,
                 precision=_PREC)
         + jnp.dot(r2, Wf1_ref[DOUT:], preferred_element_type=jnp.float32,
                   precision=_PREC)
         + bf1_ref[...])
    f = jnp.maximum(f, 0.0)
    out_ref[...] = jnp.tanh(
        jnp.dot(f, Wf2_ref[...], preferred_element_type=jnp.float32,
                precision=_PREC) + bf2_ref[...])


_head = pl.pallas_call(
    _head_body,
    out_shape=jax.ShapeDtypeStruct((1, DOUT), jnp.float32),
)


def kernel(x1, x2, edge_index1, edge_index2, W1, b1, W2, b2, W3, b3,
           Wf1, bf1, Wf2, bf2):
    src1, dst1 = edge_index1[0], edge_index1[1]
    src2, dst2 = edge_index2[0], edge_index2[1]

    ar = jnp.arange(NPAD, dtype=jnp.int32)
    gpad = (ar * 79) % N
    spad = N + (ar % PAD_ROWS)
    src1p = _pad_idx(src1, gpad)
    dst1p = _pad_idx(dst1, spad)
    src2p = _pad_idx(src2, gpad)
    dst2p = _pad_idx(dst2, spad)
    dst1g = _pad_idx(dst1, gpad)
    src1s = _pad_idx(src1, spad)
    dst2g = _pad_idx(dst2, gpad)
    src2s = _pad_idx(src2, spad)

    zeros128 = jnp.zeros((NACC, 128), jnp.float32)
    ones128 = jnp.ones((N, 128), jnp.float32)

    deg = _spmm128(ones128, ones128, src1p, src2p, dst1p, dst2p, zeros128)

    y1, y2, dinv = _prep(x1, x2, deg)
    dinv_1, dinv_2 = dinv[0], dinv[1]

    cacc = _spmm128(dinv_1, dinv_2, dst1g, dst2g, src1s, src2s, zeros128)

    p1a = _spmm128(y1[0], y1[1], src1p, src1p, dst1p, dst1p, zeros128)
    p1b = _spmm128(y2[0], y2[1], src2p, src2p, dst2p, dst2p, zeros128)

    b1r = b1.reshape(1, DH)
    y2nd1 = _mm1(p1a, y1, dinv_1, W1, b1r)
    y2nd2 = _mm1(p1b, y2, dinv_2, W1, b1r)

    p2a_lo = _spmm128(y2nd1[0], y2nd1[1], src1p, src1p, dst1p, dst1p, zeros128)
    p2a_hi = _spmm128(y2nd1[2], y2nd1[3], src1p, src1p, dst1p, dst1p, zeros128)
    p2b_lo = _spmm128(y2nd2[0], y2nd2[1], src2p, src2p, dst2p, dst2p, zeros128)
    p2b_hi = _spmm128(y2nd2[2], y2nd2[3], src2p, src2p, dst2p, dst2p, zeros128)

    b2r = b2.reshape(1, DH)
    pooled1 = _mm2(p2a_lo, p2a_hi, y2nd1, dinv_1, cacc[0], W2, b2r)
    pooled2 = _mm2(p2b_lo, p2b_hi, y2nd2, dinv_2, cacc[1], W2, b2r)

    return _head(pooled1, pooled2, W3, b3.reshape(1, DH), Wf1,
                 bf1.reshape(1, DH), Wf2, bf2.reshape(1, DOUT))


# speedup vs baseline: 17.9908x; 1.0350x over previous
import functools

import jax
import jax.numpy as jnp
from jax import lax
from jax.experimental import pallas as pl
from jax.experimental.pallas import tpu as pltpu
from jax.experimental.pallas import tpu_sc as plsc

N = 10000
E = 160000
DIN = 256
DH = 512
DOUT = 512

NC = 2
NS = 16
KW = 128
EPT = E // NS
NWIN = 80
NPH = 2
WPH = NWIN // NPH
EPTP = NWIN * KW
NPAD = EPTP - EPT
PAD_ROWS = 240
NACC = N + PAD_ROWS
RPTZ = NACC // NS


def _make_spmm(W):
    mesh = plsc.VectorSubcoreMesh(core_axis_name="c", subcore_axis_name="s")

    def body(tabA, tabB, gidxA, gidxB, sidxA, sidxB, zeros_hbm, out,
             gi_v, si_v, rows_a, rows_b, acc_sh, sem_a, sem_b):
        cid = lax.axis_index("c")
        sid = lax.axis_index("s")

        def run(tab, gidx, sidx):
            pltpu.sync_copy(zeros_hbm.at[pl.ds(sid * RPTZ, RPTZ)],
                            acc_sh.at[pl.ds(sid * RPTZ, RPTZ)])
            plsc.subcore_barrier()

            for ph in range(NPH):
                pltpu.sync_copy(gidx.at[sid, pl.ds(ph * WPH, WPH)], gi_v)
                pltpu.sync_copy(sidx.at[sid, pl.ds(ph * WPH, WPH)], si_v)

                pltpu.async_copy(tab.at[gi_v.at[0]], rows_a, sem_a)

                def win(p, carry):
                    w0 = 2 * p
                    w1 = w0 + 1
                    pltpu.async_copy(tab.at[gi_v.at[w1]], rows_b, sem_b)
                    pltpu.make_async_copy(tab.at[gi_v.at[w0]], rows_a,
                                          sem_a).wait()
                    pltpu.sync_copy(rows_a, acc_sh.at[si_v.at[w0]], add=True)

                    @pl.when(p < WPH // 2 - 1)
                    def _():
                        pltpu.async_copy(tab.at[gi_v.at[w0 + 2]], rows_a,
                                         sem_a)

                    pltpu.make_async_copy(tab.at[gi_v.at[w1]], rows_b,
                                          sem_b).wait()
                    pltpu.sync_copy(rows_b, acc_sh.at[si_v.at[w1]], add=True)
                    return carry

                lax.fori_loop(0, WPH // 2, win, 0)

            plsc.subcore_barrier()
            pltpu.sync_copy(acc_sh.at[pl.ds(sid * RPTZ, RPTZ)],
                            out.at[cid, pl.ds(sid * RPTZ, RPTZ)])

        @pl.when(cid == 0)
        def _():
            run(tabA, gidxA, sidxA)

        @pl.when(cid == 1)
        def _():
            run(tabB, gidxB, sidxB)

    return pl.kernel(
        body,
        out_type=jax.ShapeDtypeStruct((NC, NACC, W), jnp.float32),
        mesh=mesh,
        scratch_types=[
            pltpu.VMEM((WPH, KW), jnp.int32),
            pltpu.VMEM((WPH, KW), jnp.int32),
            pltpu.VMEM((KW, W), jnp.float32),
            pltpu.VMEM((KW, W), jnp.float32),
            pltpu.VMEM_SHARED((NACC, W), jnp.float32),
            pltpu.SemaphoreType.DMA,
            pltpu.SemaphoreType.DMA,
        ],
    )


_spmm128 = _make_spmm(128)


def _make_deg():
    W = 128
    mesh = plsc.VectorSubcoreMesh(core_axis_name="c", subcore_axis_name="s")

    def body(sidxA, sidxB, zeros_hbm, ones_hbm, out,
             si_v, ones_v, acc_sh, sem_a):
        cid = lax.axis_index("c")
        sid = lax.axis_index("s")

        def run(sidx):
            pltpu.sync_copy(zeros_hbm.at[pl.ds(sid * RPTZ, RPTZ)],
                            acc_sh.at[pl.ds(sid * RPTZ, RPTZ)])
            pltpu.sync_copy(ones_hbm, ones_v)
            plsc.subcore_barrier()

            for ph in range(NPH):
                pltpu.sync_copy(sidx.at[sid, pl.ds(ph * WPH, WPH)], si_v)

                def win(w, carry):
                    pltpu.sync_copy(ones_v, acc_sh.at[si_v.at[w]], add=True)
                    return carry

                lax.fori_loop(0, WPH, win, 0)

            plsc.subcore_barrier()
            pltpu.sync_copy(acc_sh.at[pl.ds(sid * RPTZ, RPTZ)],
                            out.at[cid, pl.ds(sid * RPTZ, RPTZ)])

        @pl.when(cid == 0)
        def _():
            run(sidxA)

        @pl.when(cid == 1)
        def _():
            run(sidxB)

    return pl.kernel(
        body,
        out_type=jax.ShapeDtypeStruct((NC, NACC, W), jnp.float32),
        mesh=mesh,
        scratch_types=[
            pltpu.VMEM((WPH, KW), jnp.int32),
            pltpu.VMEM((KW, W), jnp.float32),
            pltpu.VMEM_SHARED((NACC, W), jnp.float32),
            pltpu.SemaphoreType.DMA,
        ],
    )


_deg = _make_deg()


def _pad_idx(a, pad_vals):
    a = a.reshape(NS, EPT)
    pad = jnp.broadcast_to(pad_vals[None, :], (NS, NPAD))
    return jnp.concatenate([a, pad], axis=1).reshape(NS, NWIN, KW)


_PREC = lax.Precision.HIGHEST

TBP = 2000
TB = 1000


def _prep_body(x1_ref, x2_ref, deg_ref, y1_ref, y2_ref, dinv16_ref,
               dinv128_ref):
    d1 = lax.rsqrt(deg_ref[0][:, 0:1] + 1.0)
    d2 = lax.rsqrt(deg_ref[1][:, 0:1] + 1.0)
    x1 = x1_ref[...]
    x2 = x2_ref[...]
    y1_ref[0] = x1[:, :128] * d1
    y1_ref[1] = x1[:, 128:] * d1
    y2_ref[0] = x2[:, :128] * d2
    y2_ref[1] = x2[:, 128:] * d2
    dinv16_ref[0] = jnp.broadcast_to(d1, (TBP, 16))
    dinv16_ref[1] = jnp.broadcast_to(d2, (TBP, 16))
    dinv128_ref[0] = jnp.broadcast_to(d1, (TBP, 128))
    dinv128_ref[1] = jnp.broadcast_to(d2, (TBP, 128))


_prep = pl.pallas_call(
    _prep_body,
    grid=(N // TBP,),
    in_specs=[
        pl.BlockSpec((TBP, DIN), lambda i: (i, 0)),
        pl.BlockSpec((TBP, DIN), lambda i: (i, 0)),
        pl.BlockSpec((2, TBP, 128), lambda i: (0, i, 0)),
    ],
    out_specs=[
        pl.BlockSpec((2, TBP, 128), lambda i: (0, i, 0)),
        pl.BlockSpec((2, TBP, 128), lambda i: (0, i, 0)),
        pl.BlockSpec((2, TBP, 16), lambda i: (0, i, 0)),
        pl.BlockSpec((2, TBP, 128), lambda i: (0, i, 0)),
    ],
    out_shape=[
        jax.ShapeDtypeStruct((2, N, 128), jnp.float32),
        jax.ShapeDtypeStruct((2, N, 128), jnp.float32),
        jax.ShapeDtypeStruct((2, N, 16), jnp.float32),
        jax.ShapeDtypeStruct((2, N, 128), jnp.float32),
    ],
)


def _mm1_body(p1acc_ref, y_ref, dinv_ref, W1_ref, b1_ref, out_ref):
    dinv = dinv_ref[:, 0:1]
    lo = (p1acc_ref[0] + y_ref[0]) * dinv
    hi = (p1acc_ref[1] + y_ref[1]) * dinv
    P1 = jnp.concatenate([lo, hi], axis=1)
    H1 = jnp.dot(P1, W1_ref[...], preferred_element_type=jnp.float32,
                 precision=_PREC) + b1_ref[...]
    y2 = jnp.maximum(H1, 0.0) * dinv
    out_ref[0] = y2[:, 0:128]
    out_ref[1] = y2[:, 128:256]
    out_ref[2] = y2[:, 256:384]
    out_ref[3] = y2[:, 384:512]


_mm1 = pl.pallas_call(
    _mm1_body,
    grid=(N // TB,),
    in_specs=[
        pl.BlockSpec((2, TB, 128), lambda i: (0, i, 0)),
        pl.BlockSpec((2, TB, 128), lambda i: (0, i, 0)),
        pl.BlockSpec((TB, 16), lambda i: (i, 0)),
        pl.BlockSpec((DIN, DH), lambda i: (0, 0)),
        pl.BlockSpec((1, DH), lambda i: (0, 0)),
    ],
    out_specs=pl.BlockSpec((4, TB, 128), lambda i: (0, i, 0)),
    out_shape=jax.ShapeDtypeStruct((4, N, 128), jnp.float32),
)


def _mm2_body(p2lo_ref, p2hi_ref, y_ref, dinv_ref, cacc_ref, W2_ref,
              b2_ref, out_ref):
    i = pl.program_id(0)
    dinv = dinv_ref[:, 0:1]
    P2 = jnp.concatenate([
        (p2lo_ref[0] + y_ref[0]) * dinv,
        (p2lo_ref[1] + y_ref[1]) * dinv,
        (p2hi_ref[0] + y_ref[2]) * dinv,
        (p2hi_ref[1] + y_ref[3]) * dinv,
    ], axis=1)
    H2 = jnp.maximum(
        jnp.dot(P2, W2_ref[...], preferred_element_type=jnp.float32,
                precision=_PREC) + b2_ref[...], 0.0)
    c = dinv * (cacc_ref[:, 0:1] + dinv)
    part = jnp.sum(H2 * c, axis=0, keepdims=True)

    @pl.when(i == 0)
    def _():
        out_ref[...] = part

    @pl.when(i != 0)
    def _():
        out_ref[...] += part


_mm2 = pl.pallas_call(
    _mm2_body,
    grid=(N // TB,),
    in_specs=[
        pl.BlockSpec((2, TB, 128), lambda i: (0, i, 0)),
        pl.BlockSpec((2, TB, 128), lambda i: (0, i, 0)),
        pl.BlockSpec((4, TB, 128), lambda i: (0, i, 0)),
        pl.BlockSpec((TB, 16), lambda i: (i, 0)),
        pl.BlockSpec((TB, 128), lambda i: (i, 0)),
        pl.BlockSpec((DH, DH), lambda i: (0, 0)),
        pl.BlockSpec((1, DH), lambda i: (0, 0)),
    ],
    out_specs=pl.BlockSpec((1, DH), lambda i: (0, 0)),
    out_shape=jax.ShapeDtypeStruct((1, DH), jnp.float32),
)


def _head_body(s1_ref, s2_ref, W3_ref, b3_ref, Wf1_ref, bf1_ref, Wf2_ref,
               bf2_ref, out_ref):
    r1 = jnp.dot(s1_ref[...] * (1.0 / N), W3_ref[...],
                 preferred_element_type=jnp.float32, precision=_PREC) + b3_ref[...]
    r2 = jnp.dot(s2_ref[...] * (1.0 / N), W3_ref[...],
                 preferred_element_type=jnp.float32, precision=_PREC) + b3_ref[...]
    f = (jnp.dot(r1, Wf1_ref[:DOUT], preferred_element_type=jnp.float32,
                 precision=_PREC)
         + jnp.dot(r2, Wf1_ref[DOUT:], preferred_element_type=jnp.float32,
                   precision=_PREC)
         + bf1_ref[...])
    f = jnp.maximum(f, 0.0)
    out_ref[...] = jnp.tanh(
        jnp.dot(f, Wf2_ref[...], preferred_element_type=jnp.float32,
                precision=_PREC) + bf2_ref[...])


_head = pl.pallas_call(
    _head_body,
    out_shape=jax.ShapeDtypeStruct((1, DOUT), jnp.float32),
)


def kernel(x1, x2, edge_index1, edge_index2, W1, b1, W2, b2, W3, b3,
           Wf1, bf1, Wf2, bf2):
    src1, dst1 = edge_index1[0], edge_index1[1]
    src2, dst2 = edge_index2[0], edge_index2[1]

    ar = jnp.arange(NPAD, dtype=jnp.int32)
    gpad = (ar * 79) % N
    spad = N + (ar % PAD_ROWS)
    src1p = _pad_idx(src1, gpad)
    dst1p = _pad_idx(dst1, spad)
    src2p = _pad_idx(src2, gpad)
    dst2p = _pad_idx(dst2, spad)
    dst1g = _pad_idx(dst1, gpad)
    src1s = _pad_idx(src1, spad)
    dst2g = _pad_idx(dst2, gpad)
    src2s = _pad_idx(src2, spad)

    zeros128 = jnp.zeros((NACC, 128), jnp.float32)
    oneskw = jnp.ones((KW, 128), jnp.float32)

    deg = _deg(dst1p, dst2p, zeros128, oneskw)

    y1, y2, dinv16, dinv128 = _prep(x1, x2, deg)
    dinv_1, dinv_2 = dinv16[0], dinv16[1]

    cacc = _spmm128(dinv128[0], dinv128[1], dst1g, dst2g, src1s, src2s,
                    zeros128)

    p1a = _spmm128(y1[0], y1[1], src1p, src1p, dst1p, dst1p, zeros128)
    p1b = _spmm128(y2[0], y2[1], src2p, src2p, dst2p, dst2p, zeros128)

    b1r = b1.reshape(1, DH)
    y2nd1 = _mm1(p1a, y1, dinv_1, W1, b1r)
    y2nd2 = _mm1(p1b, y2, dinv_2, W1, b1r)

    p2a_lo = _spmm128(y2nd1[0], y2nd1[1], src1p, src1p, dst1p, dst1p, zeros128)
    p2a_hi = _spmm128(y2nd1[2], y2nd1[3], src1p, src1p, dst1p, dst1p, zeros128)
    p2b_lo = _spmm128(y2nd2[0], y2nd2[1], src2p, src2p, dst2p, dst2p, zeros128)
    p2b_hi = _spmm128(y2nd2[2], y2nd2[3], src2p, src2p, dst2p, dst2p, zeros128)

    b2r = b2.reshape(1, DH)
    pooled1 = _mm2(p2a_lo, p2a_hi, y2nd1, dinv_1, cacc[0], W2, b2r)
    pooled2 = _mm2(p2b_lo, p2b_hi, y2nd2, dinv_2, cacc[1], W2, b2r)

    return _head(pooled1, pooled2, W3, b3.reshape(1, DH), Wf1,
                 bf1.reshape(1, DH), Wf2, bf2.reshape(1, DOUT))


# speedup vs baseline: 18.5848x; 1.0330x over previous
import functools

import jax
import jax.numpy as jnp
from jax import lax
from jax.experimental import pallas as pl
from jax.experimental.pallas import tpu as pltpu
from jax.experimental.pallas import tpu_sc as plsc

N = 10000
E = 160000
DIN = 256
DH = 512
DOUT = 512

NC = 2
NS = 16
KW = 128
EPT = E // NS
NWIN = 80
NPH = 2
WPH = NWIN // NPH
EPTP = NWIN * KW
NPAD = EPTP - EPT
PAD_ROWS = 240
NACC = N + PAD_ROWS
RPTZ = NACC // NS


def _make_spmm(W):
    mesh = plsc.VectorSubcoreMesh(core_axis_name="c", subcore_axis_name="s")

    def body(tabA, tabB, gidxA, gidxB, sidxA, sidxB, zeros_hbm, out,
             gi_v, si_v, rows_a, rows_b, acc_sh, sem_a, sem_b):
        cid = lax.axis_index("c")
        sid = lax.axis_index("s")

        def run(tab, gidx, sidx):
            pltpu.sync_copy(zeros_hbm.at[pl.ds(sid * RPTZ, RPTZ)],
                            acc_sh.at[pl.ds(sid * RPTZ, RPTZ)])
            plsc.subcore_barrier()

            for ph in range(NPH):
                pltpu.sync_copy(gidx.at[sid, pl.ds(ph * WPH, WPH)], gi_v)
                pltpu.sync_copy(sidx.at[sid, pl.ds(ph * WPH, WPH)], si_v)

                pltpu.async_copy(tab.at[gi_v.at[0]], rows_a, sem_a)

                def win(p, carry):
                    w0 = 2 * p
                    w1 = w0 + 1
                    pltpu.async_copy(tab.at[gi_v.at[w1]], rows_b, sem_b)
                    pltpu.make_async_copy(tab.at[gi_v.at[w0]], rows_a,
                                          sem_a).wait()
                    pltpu.sync_copy(rows_a, acc_sh.at[si_v.at[w0]], add=True)

                    @pl.when(p < WPH // 2 - 1)
                    def _():
                        pltpu.async_copy(tab.at[gi_v.at[w0 + 2]], rows_a,
                                         sem_a)

                    pltpu.make_async_copy(tab.at[gi_v.at[w1]], rows_b,
                                          sem_b).wait()
                    pltpu.sync_copy(rows_b, acc_sh.at[si_v.at[w1]], add=True)
                    return carry

                lax.fori_loop(0, WPH // 2, win, 0)

            plsc.subcore_barrier()
            pltpu.sync_copy(acc_sh.at[pl.ds(sid * RPTZ, RPTZ)],
                            out.at[cid, pl.ds(sid * RPTZ, RPTZ)])

        @pl.when(cid == 0)
        def _():
            run(tabA, gidxA, sidxA)

        @pl.when(cid == 1)
        def _():
            run(tabB, gidxB, sidxB)

    return pl.kernel(
        body,
        out_type=jax.ShapeDtypeStruct((NC, NACC, W), jnp.float32),
        mesh=mesh,
        scratch_types=[
            pltpu.VMEM((WPH, KW), jnp.int32),
            pltpu.VMEM((WPH, KW), jnp.int32),
            pltpu.VMEM((KW, W), jnp.float32),
            pltpu.VMEM((KW, W), jnp.float32),
            pltpu.VMEM_SHARED((NACC, W), jnp.float32),
            pltpu.SemaphoreType.DMA,
            pltpu.SemaphoreType.DMA,
        ],
    )


_spmm128 = _make_spmm(128)


def _make_deg():
    W = 128
    mesh = plsc.VectorSubcoreMesh(core_axis_name="c", subcore_axis_name="s")

    def body(sidxA, sidxB, zeros_hbm, ones_hbm, out,
             si_v, ones_v, acc_sh, sem_a):
        cid = lax.axis_index("c")
        sid = lax.axis_index("s")

        def run(sidx):
            pltpu.sync_copy(zeros_hbm.at[pl.ds(sid * RPTZ, RPTZ)],
                            acc_sh.at[pl.ds(sid * RPTZ, RPTZ)])
            pltpu.sync_copy(ones_hbm, ones_v)
            plsc.subcore_barrier()

            for ph in range(NPH):
                pltpu.sync_copy(sidx.at[sid, pl.ds(ph * WPH, WPH)], si_v)

                def win(w, carry):
                    pltpu.sync_copy(ones_v, acc_sh.at[si_v.at[w]], add=True)
                    return carry

                lax.fori_loop(0, WPH, win, 0)

            plsc.subcore_barrier()
            pltpu.sync_copy(acc_sh.at[pl.ds(sid * RPTZ, RPTZ)],
                            out.at[cid, pl.ds(sid * RPTZ, RPTZ)])

        @pl.when(cid == 0)
        def _():
            run(sidxA)

        @pl.when(cid == 1)
        def _():
            run(sidxB)

    return pl.kernel(
        body,
        out_type=jax.ShapeDtypeStruct((NC, NACC, W), jnp.float32),
        mesh=mesh,
        scratch_types=[
            pltpu.VMEM((WPH, KW), jnp.int32),
            pltpu.VMEM((KW, W), jnp.float32),
            pltpu.VMEM_SHARED((NACC, W), jnp.float32),
            pltpu.SemaphoreType.DMA,
        ],
    )


_deg = _make_deg()


def _pad_idx(a, pad_vals):
    a = a.reshape(NS, EPT)
    pad = jnp.broadcast_to(pad_vals[None, :], (NS, NPAD))
    return jnp.concatenate([a, pad], axis=1).reshape(NS, NWIN, KW)


_PREC = lax.Precision.HIGHEST

TBP = 2000
TB = 2000


def _prep_body(x1_ref, x2_ref, deg_ref, y1a_ref, y1b_ref, y2a_ref, y2b_ref,
               d16a_ref, d16b_ref, d128a_ref, d128b_ref):
    d1 = lax.rsqrt(deg_ref[0][:, 0:1] + 1.0)
    d2 = lax.rsqrt(deg_ref[1][:, 0:1] + 1.0)
    x1 = x1_ref[...]
    x2 = x2_ref[...]
    y1a_ref[...] = x1[:, :128] * d1
    y1b_ref[...] = x1[:, 128:] * d1
    y2a_ref[...] = x2[:, :128] * d2
    y2b_ref[...] = x2[:, 128:] * d2
    d16a_ref[...] = jnp.broadcast_to(d1, (TBP, 16))
    d16b_ref[...] = jnp.broadcast_to(d2, (TBP, 16))
    d128a_ref[...] = jnp.broadcast_to(d1, (TBP, 128))
    d128b_ref[...] = jnp.broadcast_to(d2, (TBP, 128))


_prep = pl.pallas_call(
    _prep_body,
    grid=(N // TBP,),
    in_specs=[
        pl.BlockSpec((TBP, DIN), lambda i: (i, 0)),
        pl.BlockSpec((TBP, DIN), lambda i: (i, 0)),
        pl.BlockSpec((2, TBP, 128), lambda i: (0, i, 0)),
    ],
    out_specs=[
        pl.BlockSpec((TBP, 128), lambda i: (i, 0)),
        pl.BlockSpec((TBP, 128), lambda i: (i, 0)),
        pl.BlockSpec((TBP, 128), lambda i: (i, 0)),
        pl.BlockSpec((TBP, 128), lambda i: (i, 0)),
        pl.BlockSpec((TBP, 16), lambda i: (i, 0)),
        pl.BlockSpec((TBP, 16), lambda i: (i, 0)),
        pl.BlockSpec((TBP, 128), lambda i: (i, 0)),
        pl.BlockSpec((TBP, 128), lambda i: (i, 0)),
    ],
    out_shape=[jax.ShapeDtypeStruct((N, 128), jnp.float32)] * 4
              + [jax.ShapeDtypeStruct((N, 16), jnp.float32)] * 2
              + [jax.ShapeDtypeStruct((N, 128), jnp.float32)] * 2,
)


def _mm1_body(p1acc_ref, ya_ref, yb_ref, dinv_ref, W1_ref, b1_ref,
              o0_ref, o1_ref, o2_ref, o3_ref):
    dinv = dinv_ref[:, 0:1]
    lo = (p1acc_ref[0] + ya_ref[...]) * dinv
    hi = (p1acc_ref[1] + yb_ref[...]) * dinv
    P1 = jnp.concatenate([lo, hi], axis=1)
    H1 = jnp.dot(P1, W1_ref[...], preferred_element_type=jnp.float32,
                 precision=_PREC) + b1_ref[...]
    y2 = jnp.maximum(H1, 0.0) * dinv
    o0_ref[...] = y2[:, 0:128]
    o1_ref[...] = y2[:, 128:256]
    o2_ref[...] = y2[:, 256:384]
    o3_ref[...] = y2[:, 384:512]


_mm1 = pl.pallas_call(
    _mm1_body,
    grid=(N // TB,),
    in_specs=[
        pl.BlockSpec((2, TB, 128), lambda i: (0, i, 0)),
        pl.BlockSpec((TB, 128), lambda i: (i, 0)),
        pl.BlockSpec((TB, 128), lambda i: (i, 0)),
        pl.BlockSpec((TB, 16), lambda i: (i, 0)),
        pl.BlockSpec((DIN, DH), lambda i: (0, 0)),
        pl.BlockSpec((1, DH), lambda i: (0, 0)),
    ],
    out_specs=[pl.BlockSpec((TB, 128), lambda i: (i, 0))] * 4,
    out_shape=[jax.ShapeDtypeStruct((N, 128), jnp.float32)] * 4,
)


def _mm2_body(p2lo_ref, p2hi_ref, y0_ref, y1_ref, y2_ref, y3_ref,
              dinv_ref, cacc_ref, W2_ref, b2_ref, out_ref):
    i = pl.program_id(0)
    dinv = dinv_ref[:, 0:1]
    P2 = jnp.concatenate([
        (p2lo_ref[0] + y0_ref[...]) * dinv,
        (p2lo_ref[1] + y1_ref[...]) * dinv,
        (p2hi_ref[0] + y2_ref[...]) * dinv,
        (p2hi_ref[1] + y3_ref[...]) * dinv,
    ], axis=1)
    H2 = jnp.maximum(
        jnp.dot(P2, W2_ref[...], preferred_element_type=jnp.float32,
                precision=_PREC) + b2_ref[...], 0.0)
    c = dinv * (cacc_ref[0][:, 0:1] + dinv)
    part = jnp.sum(H2 * c, axis=0, keepdims=True)

    @pl.when(i == 0)
    def _():
        out_ref[...] = part

    @pl.when(i != 0)
    def _():
        out_ref[...] += part


def _make_mm2(g):
    return pl.pallas_call(
        _mm2_body,
        grid=(N // TB,),
        in_specs=[
            pl.BlockSpec((2, TB, 128), lambda i: (0, i, 0)),
            pl.BlockSpec((2, TB, 128), lambda i: (0, i, 0)),
            pl.BlockSpec((TB, 128), lambda i: (i, 0)),
            pl.BlockSpec((TB, 128), lambda i: (i, 0)),
            pl.BlockSpec((TB, 128), lambda i: (i, 0)),
            pl.BlockSpec((TB, 128), lambda i: (i, 0)),
            pl.BlockSpec((TB, 16), lambda i: (i, 0)),
            pl.BlockSpec((1, TB, 128), lambda i, g=g: (g, i, 0)),
            pl.BlockSpec((DH, DH), lambda i: (0, 0)),
            pl.BlockSpec((1, DH), lambda i: (0, 0)),
        ],
        out_specs=pl.BlockSpec((1, DH), lambda i: (0, 0)),
        out_shape=jax.ShapeDtypeStruct((1, DH), jnp.float32),
    )


_mm2_g0 = _make_mm2(0)
_mm2_g1 = _make_mm2(1)


def _head_body(s1_ref, s2_ref, W3_ref, b3_ref, Wf1_ref, bf1_ref, Wf2_ref,
               bf2_ref, out_ref):
    r1 = jnp.dot(s1_ref[...] * (1.0 / N), W3_ref[...],
                 preferred_element_type=jnp.float32, precision=_PREC) + b3_ref[...]
    r2 = jnp.dot(s2_ref[...] * (1.0 / N), W3_ref[...],
                 preferred_element_type=jnp.float32, precision=_PREC) + b3_ref[...]
    f = (jnp.dot(r1, Wf1_ref[:DOUT], preferred_element_type=jnp.float32,
                 precision=_PREC)
         + jnp.dot(r2, Wf1_ref[DOUT:], preferred_element_type=jnp.float32,
                   precision=_PREC)
         + bf1_ref[...])
    f = jnp.maximum(f, 0.0)
    out_ref[...] = jnp.tanh(
        jnp.dot(f, Wf2_ref[...], preferred_element_type=jnp.float32,
                precision=_PREC) + bf2_ref[...])


_head = pl.pallas_call(
    _head_body,
    out_shape=jax.ShapeDtypeStruct((1, DOUT), jnp.float32),
)


def kernel(x1, x2, edge_index1, edge_index2, W1, b1, W2, b2, W3, b3,
           Wf1, bf1, Wf2, bf2):
    src1, dst1 = edge_index1[0], edge_index1[1]
    src2, dst2 = edge_index2[0], edge_index2[1]

    ar = jnp.arange(NPAD, dtype=jnp.int32)
    gpad = (ar * 79) % N
    spad = N + (ar % PAD_ROWS)
    src1p = _pad_idx(src1, gpad)
    dst1p = _pad_idx(dst1, spad)
    src2p = _pad_idx(src2, gpad)
    dst2p = _pad_idx(dst2, spad)
    dst1g = _pad_idx(dst1, gpad)
    src1s = _pad_idx(src1, spad)
    dst2g = _pad_idx(dst2, gpad)
    src2s = _pad_idx(src2, spad)

    zeros128 = jnp.zeros((NACC, 128), jnp.float32)
    oneskw = jnp.ones((KW, 128), jnp.float32)

    deg = _deg(dst1p, dst2p, zeros128, oneskw)

    y1a, y1b, y2a, y2b, d16a, d16b, d128a, d128b = _prep(x1, x2, deg)

    cacc = _spmm128(d128a, d128b, dst1g, dst2g, src1s, src2s, zeros128)

    p1a = _spmm128(y1a, y1b, src1p, src1p, dst1p, dst1p, zeros128)
    p1b = _spmm128(y2a, y2b, src2p, src2p, dst2p, dst2p, zeros128)

    b1r = b1.reshape(1, DH)
    z10, z11, z12, z13 = _mm1(p1a, y1a, y1b, d16a, W1, b1r)
    z20, z21, z22, z23 = _mm1(p1b, y2a, y2b, d16b, W1, b1r)

    p2a_lo = _spmm128(z10, z11, src1p, src1p, dst1p, dst1p, zeros128)
    p2a_hi = _spmm128(z12, z13, src1p, src1p, dst1p, dst1p, zeros128)
    p2b_lo = _spmm128(z20, z21, src2p, src2p, dst2p, dst2p, zeros128)
    p2b_hi = _spmm128(z22, z23, src2p, src2p, dst2p, dst2p, zeros128)

    b2r = b2.reshape(1, DH)
    pooled1 = _mm2_g0(p2a_lo, p2a_hi, z10, z11, z12, z13, d16a, cacc, W2, b2r)
    pooled2 = _mm2_g1(p2b_lo, p2b_hi, z20, z21, z22, z23, d16b, cacc, W2, b2r)

    return _head(pooled1, pooled2, W3, b3.reshape(1, DH), Wf1,
                 bf1.reshape(1, DH), Wf2, bf2.reshape(1, DOUT))


# speedup vs baseline: 18.6273x; 1.0023x over previous
import jax
import jax.numpy as jnp
from jax import lax
from jax.experimental import pallas as pl
from jax.experimental.pallas import tpu as pltpu
from jax.experimental.pallas import tpu_sc as plsc

N = 10000
E = 160000
DIN = 256
DH = 512
DOUT = 512

NC = 2
NS = 16
KW = 128
EPT = E // NS
NWIN = 80
NPH = 2
WPH = NWIN // NPH
EPTP = NWIN * KW
NPAD = EPTP - EPT
PAD_ROWS = 240
NACC = N + PAD_ROWS
RPTZ = NACC // NS


def _make_spmm(W):
    mesh = plsc.VectorSubcoreMesh(core_axis_name="c", subcore_axis_name="s")

    def body(tabA, tabB, gidxA, gidxB, sidxA, sidxB, zeros_hbm, out,
             gi_v, si_v, rows_a, rows_b, acc_sh, sem_a, sem_b):
        cid = lax.axis_index("c")
        sid = lax.axis_index("s")

        def run(tab, gidx, sidx):
            pltpu.sync_copy(zeros_hbm.at[pl.ds(sid * RPTZ, RPTZ)],
                            acc_sh.at[pl.ds(sid * RPTZ, RPTZ)])
            plsc.subcore_barrier()

            for ph in range(NPH):
                pltpu.sync_copy(gidx.at[sid, pl.ds(ph * WPH, WPH)], gi_v)
                pltpu.sync_copy(sidx.at[sid, pl.ds(ph * WPH, WPH)], si_v)

                pltpu.async_copy(tab.at[gi_v.at[0]], rows_a, sem_a)

                def win(p, carry):
                    w0 = 2 * p
                    w1 = w0 + 1
                    pltpu.async_copy(tab.at[gi_v.at[w1]], rows_b, sem_b)
                    pltpu.make_async_copy(tab.at[gi_v.at[w0]], rows_a,
                                          sem_a).wait()
                    pltpu.sync_copy(rows_a, acc_sh.at[si_v.at[w0]], add=True)

                    @pl.when(p < WPH // 2 - 1)
                    def _():
                        pltpu.async_copy(tab.at[gi_v.at[w0 + 2]], rows_a,
                                         sem_a)

                    pltpu.make_async_copy(tab.at[gi_v.at[w1]], rows_b,
                                          sem_b).wait()
                    pltpu.sync_copy(rows_b, acc_sh.at[si_v.at[w1]], add=True)
                    return carry

                lax.fori_loop(0, WPH // 2, win, 0)

            plsc.subcore_barrier()
            pltpu.sync_copy(acc_sh.at[pl.ds(sid * RPTZ, RPTZ)],
                            out.at[cid, pl.ds(sid * RPTZ, RPTZ)])

        @pl.when(cid == 0)
        def _():
            run(tabA, gidxA, sidxA)

        @pl.when(cid == 1)
        def _():
            run(tabB, gidxB, sidxB)

    return pl.kernel(
        body,
        out_type=jax.ShapeDtypeStruct((NC, NACC, W), jnp.float32),
        mesh=mesh,
        scratch_types=[
            pltpu.VMEM((WPH, KW), jnp.int32),
            pltpu.VMEM((WPH, KW), jnp.int32),
            pltpu.VMEM((KW, W), jnp.float32),
            pltpu.VMEM((KW, W), jnp.float32),
            pltpu.VMEM_SHARED((NACC, W), jnp.float32),
            pltpu.SemaphoreType.DMA,
            pltpu.SemaphoreType.DMA,
        ],
    )


_spmm128 = _make_spmm(128)


def _make_deg():
    W = 128
    mesh = plsc.VectorSubcoreMesh(core_axis_name="c", subcore_axis_name="s")

    def body(sidxA, sidxB, zeros_hbm, ones_hbm, out,
             si_v, ones_v, acc_sh, sem_a):
        cid = lax.axis_index("c")
        sid = lax.axis_index("s")

        def run(sidx):
            pltpu.sync_copy(zeros_hbm.at[pl.ds(sid * RPTZ, RPTZ)],
                            acc_sh.at[pl.ds(sid * RPTZ, RPTZ)])
            pltpu.sync_copy(ones_hbm, ones_v)
            plsc.subcore_barrier()

            for ph in range(NPH):
                pltpu.sync_copy(sidx.at[sid, pl.ds(ph * WPH, WPH)], si_v)

                def win(w, carry):
                    pltpu.sync_copy(ones_v, acc_sh.at[si_v.at[w]], add=True)
                    return carry

                lax.fori_loop(0, WPH, win, 0)

            plsc.subcore_barrier()
            pltpu.sync_copy(acc_sh.at[pl.ds(sid * RPTZ, RPTZ)],
                            out.at[cid, pl.ds(sid * RPTZ, RPTZ)])

        @pl.when(cid == 0)
        def _():
            run(sidxA)

        @pl.when(cid == 1)
        def _():
            run(sidxB)

    return pl.kernel(
        body,
        out_type=jax.ShapeDtypeStruct((NC, NACC, W), jnp.float32),
        mesh=mesh,
        scratch_types=[
            pltpu.VMEM((WPH, KW), jnp.int32),
            pltpu.VMEM((KW, W), jnp.float32),
            pltpu.VMEM_SHARED((NACC, W), jnp.float32),
            pltpu.SemaphoreType.DMA,
        ],
    )


_deg = _make_deg()


def _pad_idx(a, pad_vals):
    a = a.reshape(NS, EPT)
    pad = jnp.broadcast_to(pad_vals[None, :], (NS, NPAD))
    return jnp.concatenate([a, pad], axis=1).reshape(NS, NWIN, KW)


_PREC = lax.Precision.HIGHEST

TBP = 2000
TB = 2000


def _prep_body(x1_ref, x2_ref, deg_ref, y1a_ref, y1b_ref, y2a_ref, y2b_ref,
               d16a_ref, d16b_ref, d128a_ref, d128b_ref):
    d1 = lax.rsqrt(deg_ref[0][:, 0:1] + 1.0)
    d2 = lax.rsqrt(deg_ref[1][:, 0:1] + 1.0)
    x1 = x1_ref[...]
    x2 = x2_ref[...]
    y1a_ref[...] = x1[:, :128] * d1
    y1b_ref[...] = x1[:, 128:] * d1
    y2a_ref[...] = x2[:, :128] * d2
    y2b_ref[...] = x2[:, 128:] * d2
    d16a_ref[...] = jnp.broadcast_to(d1, (TBP, 16))
    d16b_ref[...] = jnp.broadcast_to(d2, (TBP, 16))
    d128a_ref[...] = jnp.broadcast_to(d1, (TBP, 128))
    d128b_ref[...] = jnp.broadcast_to(d2, (TBP, 128))


_prep = pl.pallas_call(
    _prep_body,
    grid=(N // TBP,),
    in_specs=[
        pl.BlockSpec((TBP, DIN), lambda i: (i, 0)),
        pl.BlockSpec((TBP, DIN), lambda i: (i, 0)),
        pl.BlockSpec((2, TBP, 128), lambda i: (0, i, 0)),
    ],
    out_specs=[
        pl.BlockSpec((TBP, 128), lambda i: (i, 0)),
        pl.BlockSpec((TBP, 128), lambda i: (i, 0)),
        pl.BlockSpec((TBP, 128), lambda i: (i, 0)),
        pl.BlockSpec((TBP, 128), lambda i: (i, 0)),
        pl.BlockSpec((TBP, 16), lambda i: (i, 0)),
        pl.BlockSpec((TBP, 16), lambda i: (i, 0)),
        pl.BlockSpec((TBP, 128), lambda i: (i, 0)),
        pl.BlockSpec((TBP, 128), lambda i: (i, 0)),
    ],
    out_shape=[jax.ShapeDtypeStruct((N, 128), jnp.float32)] * 4
              + [jax.ShapeDtypeStruct((N, 16), jnp.float32)] * 2
              + [jax.ShapeDtypeStruct((N, 128), jnp.float32)] * 2,
)


def _mm1_body(p1acc_ref, ya_ref, yb_ref, dinv_ref, W1_ref, b1_ref,
              o0_ref, o1_ref, o2_ref, o3_ref):
    dinv = dinv_ref[:, 0:1]
    lo = (p1acc_ref[0] + ya_ref[...]) * dinv
    hi = (p1acc_ref[1] + yb_ref[...]) * dinv
    P1 = jnp.concatenate([lo, hi], axis=1)
    H1 = jnp.dot(P1, W1_ref[...], preferred_element_type=jnp.float32,
                 precision=_PREC) + b1_ref[...]
    y2 = jnp.maximum(H1, 0.0) * dinv
    o0_ref[...] = y2[:, 0:128]
    o1_ref[...] = y2[:, 128:256]
    o2_ref[...] = y2[:, 256:384]
    o3_ref[...] = y2[:, 384:512]


_mm1 = pl.pallas_call(
    _mm1_body,
    grid=(N // TB,),
    in_specs=[
        pl.BlockSpec((2, TB, 128), lambda i: (0, i, 0)),
        pl.BlockSpec((TB, 128), lambda i: (i, 0)),
        pl.BlockSpec((TB, 128), lambda i: (i, 0)),
        pl.BlockSpec((TB, 16), lambda i: (i, 0)),
        pl.BlockSpec((DIN, DH), lambda i: (0, 0)),
        pl.BlockSpec((1, DH), lambda i: (0, 0)),
    ],
    out_specs=[pl.BlockSpec((TB, 128), lambda i: (i, 0))] * 4,
    out_shape=[jax.ShapeDtypeStruct((N, 128), jnp.float32)] * 4,
)


def _mm2_body(p2lo_ref, p2hi_ref, y0_ref, y1_ref, y2_ref, y3_ref,
              dinv_ref, cacc_ref, W2_ref, b2_ref, out_ref):
    i = pl.program_id(0)
    dinv = dinv_ref[:, 0:1]
    P2 = jnp.concatenate([
        (p2lo_ref[0] + y0_ref[...]) * dinv,
        (p2lo_ref[1] + y1_ref[...]) * dinv,
        (p2hi_ref[0] + y2_ref[...]) * dinv,
        (p2hi_ref[1] + y3_ref[...]) * dinv,
    ], axis=1)
    H2 = jnp.maximum(
        jnp.dot(P2, W2_ref[...], preferred_element_type=jnp.float32,
                precision=_PREC) + b2_ref[...], 0.0)
    c = dinv * (cacc_ref[0][:, 0:1] + dinv)
    part = jnp.sum(H2 * c, axis=0, keepdims=True)

    @pl.when(i == 0)
    def _():
        out_ref[...] = part

    @pl.when(i != 0)
    def _():
        out_ref[...] += part


def _make_mm2(g):
    return pl.pallas_call(
        _mm2_body,
        grid=(N // TB,),
        in_specs=[
            pl.BlockSpec((2, TB, 128), lambda i: (0, i, 0)),
            pl.BlockSpec((2, TB, 128), lambda i: (0, i, 0)),
            pl.BlockSpec((TB, 128), lambda i: (i, 0)),
            pl.BlockSpec((TB, 128), lambda i: (i, 0)),
            pl.BlockSpec((TB, 128), lambda i: (i, 0)),
            pl.BlockSpec((TB, 128), lambda i: (i, 0)),
            pl.BlockSpec((TB, 16), lambda i: (i, 0)),
            pl.BlockSpec((1, TB, 128), lambda i, g=g: (g, i, 0)),
            pl.BlockSpec((DH, DH), lambda i: (0, 0)),
            pl.BlockSpec((1, DH), lambda i: (0, 0)),
        ],
        out_specs=pl.BlockSpec((1, DH), lambda i: (0, 0)),
        out_shape=jax.ShapeDtypeStruct((1, DH), jnp.float32),
    )


_mm2_g0 = _make_mm2(0)
_mm2_g1 = _make_mm2(1)


def _head_body(s1_ref, s2_ref, W3_ref, b3_ref, Wf1_ref, bf1_ref, Wf2_ref,
               bf2_ref, out_ref):
    r1 = jnp.dot(s1_ref[...] * (1.0 / N), W3_ref[...],
                 preferred_element_type=jnp.float32, precision=_PREC) + b3_ref[...]
    r2 = jnp.dot(s2_ref[...] * (1.0 / N), W3_ref[...],
                 preferred_element_type=jnp.float32, precision=_PREC) + b3_ref[...]
    f = (jnp.dot(r1, Wf1_ref[:DOUT], preferred_element_type=jnp.float32,
                 precision=_PREC)
         + jnp.dot(r2, Wf1_ref[DOUT:], preferred_element_type=jnp.float32,
                   precision=_PREC)
         + bf1_ref[...])
    f = jnp.maximum(f, 0.0)
    out_ref[...] = jnp.tanh(
        jnp.dot(f, Wf2_ref[...], preferred_element_type=jnp.float32,
                precision=_PREC) + bf2_ref[...])


_head = pl.pallas_call(
    _head_body,
    out_shape=jax.ShapeDtypeStruct((1, DOUT), jnp.float32),
)


def kernel(x1, x2, edge_index1, edge_index2, W1, b1, W2, b2, W3, b3,
           Wf1, bf1, Wf2, bf2):
    src1, dst1 = edge_index1[0], edge_index1[1]
    src2, dst2 = edge_index2[0], edge_index2[1]

    ar = jnp.arange(NPAD, dtype=jnp.int32)
    gpad = (ar * 79) % N
    spad = N + (ar % PAD_ROWS)
    src1p = _pad_idx(src1, gpad)
    dst1p = _pad_idx(dst1, spad)
    src2p = _pad_idx(src2, gpad)
    dst2p = _pad_idx(dst2, spad)
    dst1g = _pad_idx(dst1, gpad)
    src1s = _pad_idx(src1, spad)
    dst2g = _pad_idx(dst2, gpad)
    src2s = _pad_idx(src2, spad)

    zeros128 = jnp.zeros((NACC, 128), jnp.float32)
    oneskw = jnp.ones((KW, 128), jnp.float32)

    deg = _deg(dst1p, dst2p, zeros128, oneskw)

    y1a, y1b, y2a, y2b, d16a, d16b, d128a, d128b = _prep(x1, x2, deg)

    cacc = _spmm128(d128a, d128b, dst1g, dst2g, src1s, src2s, zeros128)

    p1a = _spmm128(y1a, y1b, src1p, src1p, dst1p, dst1p, zeros128)
    p1b = _spmm128(y2a, y2b, src2p, src2p, dst2p, dst2p, zeros128)

    b1r = b1.reshape(1, DH)
    z10, z11, z12, z13 = _mm1(p1a, y1a, y1b, d16a, W1, b1r)
    z20, z21, z22, z23 = _mm1(p1b, y2a, y2b, d16b, W1, b1r)

    p2a_lo = _spmm128(z10, z11, src1p, src1p, dst1p, dst1p, zeros128)
    p2a_hi = _spmm128(z12, z13, src1p, src1p, dst1p, dst1p, zeros128)
    p2b_lo = _spmm128(z20, z21, src2p, src2p, dst2p, dst2p, zeros128)
    p2b_hi = _spmm128(z22, z23, src2p, src2p, dst2p, dst2p, zeros128)

    b2r = b2.reshape(1, DH)
    pooled1 = _mm2_g0(p2a_lo, p2a_hi, z10, z11, z12, z13, d16a, cacc, W2, b2r)
    pooled2 = _mm2_g1(p2b_lo, p2b_hi, z20, z21, z22, z23, d16b, cacc, W2, b2r)

    return _head(pooled1, pooled2, W3, b3.reshape(1, DH), Wf1,
                 bf1.reshape(1, DH), Wf2, bf2.reshape(1, DOUT))
